# baseline (speedup 1.0000x reference)
import math

import jax
import jax.numpy as jnp
from jax import lax
from jax.experimental import pallas as pl
from jax.experimental.pallas import tpu as pltpu
from jax.experimental.pallas import tpu_sc as plsc

EMBED_DIM = 64
SEQ_LEN = 200
BATCH = 1024
ROWS = BATCH * SEQ_LEN
NC, NS, LANES = 2, 16, 16
NW = NC * NS
RPW = ROWS // NW
CHUNK = 128
NCHUNK = RPW // CHUNK
NBUF = 2
PE_EXT = SEQ_LEN + CHUNK
SCALE = math.sqrt(EMBED_DIM)


def _positional_table():
    pe_len = SEQ_LEN * 2
    pos = jnp.arange(pe_len, dtype=jnp.float32)[:, None]
    i = jnp.arange(0, EMBED_DIM, 2, dtype=jnp.float32)[None, :]
    sin_part = jnp.sin(pos / jnp.power(10000.0, 2.0 * i / EMBED_DIM))
    cos_part = jnp.cos(pos / jnp.power(10000.0, 2.0 * (i + 1.0) / EMBED_DIM))
    pe = jnp.zeros((pe_len, EMBED_DIM), dtype=jnp.float32)
    pe = pe.at[:, 0::2].set(sin_part)
    pe = pe.at[:, 1::2].set(cos_part)
    pe = pe[:SEQ_LEN]
    return jnp.concatenate([pe, pe[: PE_EXT - SEQ_LEN]], axis=0)


def _sc_body(idx_hbm, par_hbm, pe_hbm, table_hbm, out_hbm,
             idx_v, par_v, pe_v, *rest):
    bufs = rest[:NBUF]
    obufs = rest[NBUF:2 * NBUF]
    sg = rest[2 * NBUF:3 * NBUF]
    so = rest[3 * NBUF:]
    cid = lax.axis_index("c")
    sid = lax.axis_index("s")
    wid = sid * NC + cid
    out_base = wid * (RPW // 2)

    pltpu.sync_copy(idx_hbm.at[wid], idx_v)
    pltpu.sync_copy(par_hbm.at[wid], par_v)
    pltpu.sync_copy(pe_hbm, pe_v)

    for b in range(NBUF - 1):
        pltpu.async_copy(table_hbm.at[idx_v.at[b]], bufs[b], sg[b])

    def group(g, carry):
        for b in range(NBUF):
            j = g * NBUF + b
            buf, obuf = bufs[b], obufs[b]
            pltpu.make_async_copy(table_hbm.at[idx_v.at[j]], buf, sg[b]).wait()

            p0 = (j * CHUNK) % SEQ_LEN

            @plsc.parallel_loop(0, CHUNK, step=LANES)
            def _row(i):
                parv = par_v[j, pl.ds(i, LANES)]
                i2 = i // 2
                for r in range(LANES):
                    off = parv[r]
                    for k in range(EMBED_DIM // LANES):
                        src = buf[i + r, pl.ds(off + k * LANES, LANES)]
                        pe_part = pe_v[p0 + i + r, pl.ds(k * LANES, LANES)]
                        dst = pl.ds((r % 2) * EMBED_DIM + k * LANES, LANES)
                        obuf[i2 + r // 2, dst] = src * SCALE + pe_part

            pltpu.async_copy(
                obuf,
                out_hbm.at[pl.ds(out_base + j * (CHUNK // 2), CHUNK // 2)],
                so[b])

            bn = (b - 1) % NBUF

            def _retire():
                pltpu.make_async_copy(
                    obufs[bn],
                    out_hbm.at[pl.ds(0, CHUNK // 2)], so[bn]).wait()

            def _refill():
                jn = j + NBUF - 1
                pltpu.async_copy(table_hbm.at[idx_v.at[jn]], bufs[bn], sg[bn])

            if b == 0:
                pl.when(g >= 1)(_retire)
                _refill()
            else:
                _retire()
                pl.when(g * NBUF + b + NBUF - 1 < NCHUNK)(_refill)
        return carry

    lax.fori_loop(0, NCHUNK // NBUF, group, 0)

    bl = (NCHUNK - 1) % NBUF
    pltpu.make_async_copy(
        obufs[bl], out_hbm.at[pl.ds(0, CHUNK // 2)], so[bl]).wait()


_emb = pl.kernel(
    _sc_body,
    out_type=jax.ShapeDtypeStruct((ROWS // 2, 128), jnp.float32),
    mesh=plsc.VectorSubcoreMesh(
        core_axis_name="c", subcore_axis_name="s",
        num_cores=NC, num_subcores=NS,
    ),
    scratch_types=(
        [pltpu.VMEM((NCHUNK, CHUNK), jnp.int32),
         pltpu.VMEM((NCHUNK, CHUNK), jnp.int32),
         pltpu.VMEM((PE_EXT, EMBED_DIM), jnp.float32)]
        + [pltpu.VMEM((CHUNK, 128), jnp.float32) for _ in range(NBUF)]
        + [pltpu.VMEM((CHUNK // 2, 128), jnp.float32) for _ in range(NBUF)]
        + [pltpu.SemaphoreType.DMA for _ in range(2 * NBUF)]
    ),
)


def kernel(input_seq, embed_weight):
    idx = input_seq.astype(jnp.int32)
    pair = (idx // 2).reshape(NW, NCHUNK, CHUNK)
    par = ((idx % 2) * EMBED_DIM).reshape(NW, NCHUNK, CHUNK)
    table2 = embed_weight.reshape(500000, 128)
    out = _emb(pair, par, _positional_table(), table2)
    return out.reshape(BATCH, SEQ_LEN, EMBED_DIM)
